# baseline (speedup 1.0000x reference)
import functools

import jax
import jax.numpy as jnp
from jax import lax
from jax.experimental import pallas as pl
from jax.experimental.pallas import tpu as pltpu
from jax.experimental.pallas import tpu_sc as plsc

N_CH = 16
BATCH = 4096
MAX_LEN = 200
NW = 32
COLS_W = BATCH // NW
ROWS_C = 8
R_SC = 48
R_TC = MAX_LEN - R_SC
NCHUNK = R_SC // ROWS_C
NPAIR = NCHUNK // 2
TC_ROWS_B = 8
TABLE_MAX_KEY = 16


def _sc_body(in_hbm, out_hbm, buf, obuf, sin0, sin1, sout0, sout1):
    core = lax.axis_index("c")
    sub = lax.axis_index("s")
    wid = sub * 2 + core
    col0 = wid * COLS_W
    sins = (sin0, sin1)
    souts = (sout0, sout1)

    def in_copies(ci, b):
        r = ci * ROWS_C
        return [
            pltpu.make_async_copy(
                in_hbm.at[c, pl.ds(r, ROWS_C), pl.ds(col0, COLS_W)],
                buf.at[b, c],
                sins[b],
            )
            for c in range(N_CH)
        ]

    def out_copy(ci, b):
        r = ci * ROWS_C
        return pltpu.make_async_copy(
            obuf.at[b],
            out_hbm.at[pl.ds(r, ROWS_C), pl.ds(col0, COLS_W)],
            souts[b],
        )

    def compute(b):
        def row_body(r, carry):
            for o in range(0, COLS_W, 16):
                acc = buf[b, 0, r, pl.ds(o, 16)]
                for c in range(1, N_CH):
                    acc = acc + buf[b, c, r, pl.ds(o, 16)] * jnp.int32(c + 1)
                u = lax.bitcast_convert_type(acc, jnp.uint32)
                acc = jnp.where(
                    u <= jnp.uint32(TABLE_MAX_KEY), acc, jnp.int32(0)
                )
                obuf[b, r, pl.ds(o, 16)] = acc
            return carry

        lax.fori_loop(0, ROWS_C, row_body, jnp.int32(0))

    for cp in in_copies(0, 0):
        cp.start()
    for cp in in_copies(1, 1):
        cp.start()

    def pair_body(j, carry):
        for b in (0, 1):
            ci = 2 * j + b
            for cp in in_copies(ci, b):
                cp.wait()

            @pl.when(j > 0)
            def _():
                out_copy(ci, b).wait()

            compute(b)
            out_copy(ci, b).start()

            @pl.when(j < NPAIR - 1)
            def _():
                for cp in in_copies(ci + 2, b):
                    cp.start()

        return carry

    lax.fori_loop(0, NPAIR, pair_body, jnp.int32(0))
    out_copy(NCHUNK - 2, 0).wait()
    out_copy(NCHUNK - 1, 1).wait()


_mesh = plsc.VectorSubcoreMesh(core_axis_name="c", subcore_axis_name="s")

_sc_call = functools.partial(
    pl.kernel,
    mesh=_mesh,
    out_type=jax.ShapeDtypeStruct((R_SC, BATCH), jnp.int32),
    scratch_types=[
        pltpu.VMEM((2, N_CH, ROWS_C, COLS_W), jnp.int32),
        pltpu.VMEM((2, ROWS_C, COLS_W), jnp.int32),
        pltpu.SemaphoreType.DMA,
        pltpu.SemaphoreType.DMA,
        pltpu.SemaphoreType.DMA,
        pltpu.SemaphoreType.DMA,
    ],
)(_sc_body)


def _tc_body(in_ref, out_ref):
    x = in_ref[...]
    acc = x[0]
    for c in range(1, N_CH):
        acc = acc + x[c] * jnp.int32(c + 1)
    u = lax.bitcast_convert_type(acc, jnp.uint32)
    out_ref[...] = jnp.where(
        u <= jnp.uint32(TABLE_MAX_KEY), acc, jnp.int32(0)
    )


_tc_call = pl.pallas_call(
    _tc_body,
    grid=(R_TC // TC_ROWS_B,),
    in_specs=[
        pl.BlockSpec(
            (N_CH, TC_ROWS_B, BATCH),
            lambda i: (0, R_SC // TC_ROWS_B + i, 0),
        )
    ],
    out_specs=pl.BlockSpec((TC_ROWS_B, BATCH), lambda i: (i, 0)),
    out_shape=jax.ShapeDtypeStruct((R_TC, BATCH), jnp.int32),
)


TC_ROWS_FULL = 40


def _tc_acc_body(in_ref, out_ref, acc_ref):
    c = pl.program_id(1)
    x = in_ref[0]
    w = (c + 1).astype(jnp.int32)

    @pl.when(c == 0)
    def _():
        acc_ref[...] = x

    @pl.when(c > 0)
    def _():
        acc_ref[...] = acc_ref[...] + x * w

    @pl.when(c == N_CH - 1)
    def _():
        acc = acc_ref[...]
        u = lax.bitcast_convert_type(acc, jnp.uint32)
        out_ref[...] = jnp.where(
            u <= jnp.uint32(TABLE_MAX_KEY), acc, jnp.int32(0)
        )


_tc_call_full = pl.pallas_call(
    _tc_acc_body,
    grid=(MAX_LEN // TC_ROWS_FULL, N_CH),
    in_specs=[
        pl.BlockSpec(
            (1, TC_ROWS_FULL, BATCH),
            lambda i, c: (c, i, 0),
        )
    ],
    out_specs=pl.BlockSpec((TC_ROWS_FULL, BATCH), lambda i, c: (i, 0)),
    out_shape=jax.ShapeDtypeStruct((MAX_LEN, BATCH), jnp.int32),
    scratch_shapes=[pltpu.VMEM((TC_ROWS_FULL, BATCH), jnp.int32)],
)


def kernel(inputs):
    transposed = jnp.transpose(inputs, (0, 2, 1))
    return _tc_call_full(transposed).T


# speedup vs baseline: 1.3363x; 1.3363x over previous
import functools

import jax
import jax.numpy as jnp
from jax import lax
from jax.experimental import pallas as pl
from jax.experimental.pallas import tpu as pltpu
from jax.experimental.pallas import tpu_sc as plsc

N_CH = 16
BATCH = 4096
MAX_LEN = 200
NW = 32
COLS_W = BATCH // NW
ROWS_C = 8
R_SC = 80
R_TC = MAX_LEN - R_SC
NCHUNK = R_SC // ROWS_C
NPAIR = NCHUNK // 2
TC_ROWS_B = 40
TABLE_MAX_KEY = 16


def _sc_body(in_hbm, out_hbm, buf, obuf, sin0, sin1, sout0, sout1):
    core = lax.axis_index("c")
    sub = lax.axis_index("s")
    wid = sub * 2 + core
    col0 = wid * COLS_W
    sins = (sin0, sin1)
    souts = (sout0, sout1)

    def in_copies(ci, b):
        r = ci * ROWS_C
        return [
            pltpu.make_async_copy(
                in_hbm.at[c, pl.ds(r, ROWS_C), pl.ds(col0, COLS_W)],
                buf.at[b, c],
                sins[b],
            )
            for c in range(N_CH)
        ]

    def out_copy(ci, b):
        r = ci * ROWS_C
        return pltpu.make_async_copy(
            obuf.at[b],
            out_hbm.at[pl.ds(r, ROWS_C), pl.ds(col0, COLS_W)],
            souts[b],
        )

    def compute(b):
        def row_body(r, carry):
            for o in range(0, COLS_W, 16):
                acc = buf[b, 0, r, pl.ds(o, 16)]
                for c in range(1, N_CH):
                    acc = acc + buf[b, c, r, pl.ds(o, 16)] * jnp.int32(c + 1)
                u = lax.bitcast_convert_type(acc, jnp.uint32)
                acc = jnp.where(
                    u <= jnp.uint32(TABLE_MAX_KEY), acc, jnp.int32(0)
                )
                obuf[b, r, pl.ds(o, 16)] = acc
            return carry

        lax.fori_loop(0, ROWS_C, row_body, jnp.int32(0))

    for cp in in_copies(0, 0):
        cp.start()
    for cp in in_copies(1, 1):
        cp.start()

    def pair_body(j, carry):
        for b in (0, 1):
            ci = 2 * j + b
            for cp in in_copies(ci, b):
                cp.wait()

            @pl.when(j > 0)
            def _():
                out_copy(ci, b).wait()

            compute(b)
            out_copy(ci, b).start()

            @pl.when(j < NPAIR - 1)
            def _():
                for cp in in_copies(ci + 2, b):
                    cp.start()

        return carry

    lax.fori_loop(0, NPAIR, pair_body, jnp.int32(0))
    out_copy(NCHUNK - 2, 0).wait()
    out_copy(NCHUNK - 1, 1).wait()


_mesh = plsc.VectorSubcoreMesh(core_axis_name="c", subcore_axis_name="s")

_sc_call = functools.partial(
    pl.kernel,
    mesh=_mesh,
    out_type=jax.ShapeDtypeStruct((R_SC, BATCH), jnp.int32),
    scratch_types=[
        pltpu.VMEM((2, N_CH, ROWS_C, COLS_W), jnp.int32),
        pltpu.VMEM((2, ROWS_C, COLS_W), jnp.int32),
        pltpu.SemaphoreType.DMA,
        pltpu.SemaphoreType.DMA,
        pltpu.SemaphoreType.DMA,
        pltpu.SemaphoreType.DMA,
    ],
)(_sc_body)


def _tc_body(in_ref, out_ref):
    x = in_ref[...]
    acc = x[0]
    for c in range(1, N_CH):
        acc = acc + x[c] * jnp.int32(c + 1)
    u = lax.bitcast_convert_type(acc, jnp.uint32)
    out_ref[...] = jnp.where(
        u <= jnp.uint32(TABLE_MAX_KEY), acc, jnp.int32(0)
    )


_tc_call = pl.pallas_call(
    _tc_body,
    grid=(R_TC // TC_ROWS_B,),
    in_specs=[
        pl.BlockSpec(
            (N_CH, TC_ROWS_B, BATCH),
            lambda i: (0, R_SC // TC_ROWS_B + i, 0),
        )
    ],
    out_specs=pl.BlockSpec(
        (TC_ROWS_B, BATCH), lambda i: (R_SC // TC_ROWS_B + i, 0)
    ),
    out_shape=jax.ShapeDtypeStruct((MAX_LEN, BATCH), jnp.int32),
)


def kernel(inputs):
    transposed = jnp.transpose(inputs, (0, 2, 1))
    top = _sc_call(transposed)
    full = _tc_call(transposed)
    return lax.dynamic_update_slice(full, top, (0, 0)).T


# speedup vs baseline: 1.3957x; 1.0445x over previous
import functools

import jax
import jax.numpy as jnp
from jax import lax
from jax.experimental import pallas as pl
from jax.experimental.pallas import tpu as pltpu
from jax.experimental.pallas import tpu_sc as plsc

N_CH = 16
BATCH = 4096
MAX_LEN = 200
NW = 32
COLS_W = BATCH // NW
ROWS_C = 8
R_SC = 56
R_TC = MAX_LEN - R_SC
NCHUNK = R_SC // ROWS_C
NPAIR = NCHUNK // 2
TC_ROWS_B = 24
TABLE_MAX_KEY = 16


def _sc_body(in_hbm, out_hbm, buf, obuf, sin0, sin1, sout0, sout1):
    core = lax.axis_index("c")
    sub = lax.axis_index("s")
    wid = sub * 2 + core
    col0 = wid * COLS_W
    sins = (sin0, sin1)
    souts = (sout0, sout1)

    def in_copies(ci, b):
        r = ci * ROWS_C
        return [
            pltpu.make_async_copy(
                in_hbm.at[c, pl.ds(r, ROWS_C), pl.ds(col0, COLS_W)],
                buf.at[b, c],
                sins[b],
            )
            for c in range(N_CH)
        ]

    def out_copy(ci, b):
        r = ci * ROWS_C
        return pltpu.make_async_copy(
            obuf.at[b],
            out_hbm.at[pl.ds(r, ROWS_C), pl.ds(col0, COLS_W)],
            souts[b],
        )

    def compute(b):
        def row_body(r, carry):
            for o in range(0, COLS_W, 16):
                acc = buf[b, 0, r, pl.ds(o, 16)]
                for c in range(1, N_CH):
                    acc = acc + buf[b, c, r, pl.ds(o, 16)] * jnp.int32(c + 1)
                u = lax.bitcast_convert_type(acc, jnp.uint32)
                acc = jnp.where(
                    u <= jnp.uint32(TABLE_MAX_KEY), acc, jnp.int32(0)
                )
                obuf[b, r, pl.ds(o, 16)] = acc
            return carry

        lax.fori_loop(0, ROWS_C, row_body, jnp.int32(0))

    for cp in in_copies(0, 0):
        cp.start()
    for cp in in_copies(1, 1):
        cp.start()

    def pair_body(j, carry):
        for b in (0, 1):
            ci = 2 * j + b
            for cp in in_copies(ci, b):
                cp.wait()

            @pl.when(j > 0)
            def _():
                out_copy(ci, b).wait()

            compute(b)
            out_copy(ci, b).start()

            last_pf = NPAIR if (NCHUNK % 2 and b == 0) else NPAIR - 1

            @pl.when(j < last_pf)
            def _():
                for cp in in_copies(ci + 2, b):
                    cp.start()

        return carry

    lax.fori_loop(0, NPAIR, pair_body, jnp.int32(0))

    if NCHUNK % 2:
        last = NCHUNK - 1
        for cp in in_copies(last, 0):
            cp.wait()
        out_copy(last, 0).wait()
        compute(0)
        out_copy(last, 0).start()
        out_copy(last - 1, 1).wait()
        out_copy(last, 0).wait()
    else:
        out_copy(NCHUNK - 2, 0).wait()
        out_copy(NCHUNK - 1, 1).wait()


_mesh = plsc.VectorSubcoreMesh(core_axis_name="c", subcore_axis_name="s")

_sc_call = functools.partial(
    pl.kernel,
    mesh=_mesh,
    out_type=jax.ShapeDtypeStruct((R_SC, BATCH), jnp.int32),
    scratch_types=[
        pltpu.VMEM((2, N_CH, ROWS_C, COLS_W), jnp.int32),
        pltpu.VMEM((2, ROWS_C, COLS_W), jnp.int32),
        pltpu.SemaphoreType.DMA,
        pltpu.SemaphoreType.DMA,
        pltpu.SemaphoreType.DMA,
        pltpu.SemaphoreType.DMA,
    ],
)(_sc_body)


def _tc_body(in_ref, out_ref):
    x = in_ref[...]
    acc = x[0]
    for c in range(1, N_CH):
        acc = acc + x[c] * jnp.int32(c + 1)
    u = lax.bitcast_convert_type(acc, jnp.uint32)
    out_ref[...] = jnp.where(
        u <= jnp.uint32(TABLE_MAX_KEY), acc, jnp.int32(0)
    )


_tc_call = pl.pallas_call(
    _tc_body,
    grid=(R_TC // TC_ROWS_B,),
    in_specs=[
        pl.BlockSpec(
            (N_CH, TC_ROWS_B, BATCH),
            lambda i: (0, R_SC // TC_ROWS_B + i, 0),
        )
    ],
    out_specs=pl.BlockSpec(
        (TC_ROWS_B, BATCH), lambda i: (R_SC // TC_ROWS_B + i, 0)
    ),
    out_shape=jax.ShapeDtypeStruct((MAX_LEN, BATCH), jnp.int32),
)


def kernel(inputs):
    transposed = jnp.transpose(inputs, (0, 2, 1))
    top = _sc_call(transposed)
    full = _tc_call(transposed)
    return lax.dynamic_update_slice(full, top, (0, 0)).T


# speedup vs baseline: 1.4622x; 1.0477x over previous
import functools

import jax
import jax.numpy as jnp
from jax import lax
from jax.experimental import pallas as pl
from jax.experimental.pallas import tpu as pltpu
from jax.experimental.pallas import tpu_sc as plsc

N_CH = 16
BATCH = 4096
MAX_LEN = 200
NW = 32
COLS_W = BATCH // NW
ROWS_C = 8
R_SC = 40
R_TC = MAX_LEN - R_SC
NCHUNK = R_SC // ROWS_C
NPAIR = NCHUNK // 2
TC_ROWS_B = 40
TABLE_MAX_KEY = 16


def _sc_body(in_hbm, out_hbm, buf, obuf, sin0, sin1, sout0, sout1):
    core = lax.axis_index("c")
    sub = lax.axis_index("s")
    wid = sub * 2 + core
    col0 = wid * COLS_W
    sins = (sin0, sin1)
    souts = (sout0, sout1)

    def in_copies(ci, b):
        r = ci * ROWS_C
        return [
            pltpu.make_async_copy(
                in_hbm.at[c, pl.ds(r, ROWS_C), pl.ds(col0, COLS_W)],
                buf.at[b, c],
                sins[b],
            )
            for c in range(N_CH)
        ]

    def out_copy(ci, b):
        r = ci * ROWS_C
        return pltpu.make_async_copy(
            obuf.at[b],
            out_hbm.at[pl.ds(r, ROWS_C), pl.ds(col0, COLS_W)],
            souts[b],
        )

    def compute(b):
        def row_body(r, carry):
            for o in range(0, COLS_W, 16):
                acc = buf[b, 0, r, pl.ds(o, 16)]
                for c in range(1, N_CH):
                    acc = acc + buf[b, c, r, pl.ds(o, 16)] * jnp.int32(c + 1)
                u = lax.bitcast_convert_type(acc, jnp.uint32)
                acc = jnp.where(
                    u <= jnp.uint32(TABLE_MAX_KEY), acc, jnp.int32(0)
                )
                obuf[b, r, pl.ds(o, 16)] = acc
            return carry

        lax.fori_loop(0, ROWS_C, row_body, jnp.int32(0))

    for cp in in_copies(0, 0):
        cp.start()
    for cp in in_copies(1, 1):
        cp.start()

    def pair_body(j, carry):
        for b in (0, 1):
            ci = 2 * j + b
            for cp in in_copies(ci, b):
                cp.wait()

            @pl.when(j > 0)
            def _():
                out_copy(ci, b).wait()

            compute(b)
            out_copy(ci, b).start()

            last_pf = NPAIR if (NCHUNK % 2 and b == 0) else NPAIR - 1

            @pl.when(j < last_pf)
            def _():
                for cp in in_copies(ci + 2, b):
                    cp.start()

        return carry

    lax.fori_loop(0, NPAIR, pair_body, jnp.int32(0))

    if NCHUNK % 2:
        last = NCHUNK - 1
        for cp in in_copies(last, 0):
            cp.wait()
        out_copy(last, 0).wait()
        compute(0)
        out_copy(last, 0).start()
        out_copy(last - 1, 1).wait()
        out_copy(last, 0).wait()
    else:
        out_copy(NCHUNK - 2, 0).wait()
        out_copy(NCHUNK - 1, 1).wait()


_mesh = plsc.VectorSubcoreMesh(core_axis_name="c", subcore_axis_name="s")

_sc_call = functools.partial(
    pl.kernel,
    mesh=_mesh,
    out_type=jax.ShapeDtypeStruct((R_SC, BATCH), jnp.int32),
    scratch_types=[
        pltpu.VMEM((2, N_CH, ROWS_C, COLS_W), jnp.int32),
        pltpu.VMEM((2, ROWS_C, COLS_W), jnp.int32),
        pltpu.SemaphoreType.DMA,
        pltpu.SemaphoreType.DMA,
        pltpu.SemaphoreType.DMA,
        pltpu.SemaphoreType.DMA,
    ],
)(_sc_body)


def _tc_body(in_ref, out_ref):
    x = in_ref[...]
    acc = x[0]
    for c in range(1, N_CH):
        acc = acc + x[c] * jnp.int32(c + 1)
    u = lax.bitcast_convert_type(acc, jnp.uint32)
    out_ref[...] = jnp.where(
        u <= jnp.uint32(TABLE_MAX_KEY), acc, jnp.int32(0)
    )


_tc_call = pl.pallas_call(
    _tc_body,
    grid=(R_TC // TC_ROWS_B,),
    in_specs=[
        pl.BlockSpec(
            (N_CH, TC_ROWS_B, BATCH),
            lambda i: (0, R_SC // TC_ROWS_B + i, 0),
        )
    ],
    out_specs=pl.BlockSpec(
        (TC_ROWS_B, BATCH), lambda i: (R_SC // TC_ROWS_B + i, 0)
    ),
    out_shape=jax.ShapeDtypeStruct((MAX_LEN, BATCH), jnp.int32),
)


def kernel(inputs):
    transposed = jnp.transpose(inputs, (0, 2, 1))
    top = _sc_call(transposed)
    full = _tc_call(transposed)
    return lax.dynamic_update_slice(full, top, (0, 0)).T
